# baseline (speedup 1.0000x reference)
import jax
import jax.numpy as jnp
from jax import lax
from jax.experimental import pallas as pl
from jax.experimental.pallas import tpu as pltpu
from jax.experimental.pallas import tpu_sc as plsc

_B, _N, _C, _NCLS = 64, 196, 384, 1000
_K = int((1.0 - 0.7) * _N)
_MININT = -2147483648

_B_SC = 16
_R_SC = _B_SC * _N
_B_TC = _B - _B_SC
_BB = 8
_G = _B_TC // _BB
_CHUNKS = [(r0, 16) for r0 in range(0, 192, 16)] + [(192, 4)]


def _sc_body(sf_hbm, sft_hbm, sp_hbm, st_hbm, dt_hbm,
             xbuf0, xtbuf0, xbuf1, xtbuf1, x4buf, xt4buf,
             spbuf, stbuf, dtbuf, sem0, sem1):
    nc = 2
    wid = lax.axis_index("s") * nc + lax.axis_index("c")
    bufs = [(xbuf0, xtbuf0, sem0), (xbuf1, xtbuf1, sem1)]

    @pl.when(wid < _B_SC)
    def _active():
        _sc_tile(sf_hbm, sft_hbm, sp_hbm, st_hbm, dt_hbm, wid, bufs,
                 x4buf, xt4buf, spbuf, stbuf, dtbuf)


def _sc_tile(sf_hbm, sft_hbm, sp_hbm, st_hbm, dt_hbm, wid, bufs,
             x4buf, xt4buf, spbuf, stbuf, dtbuf):

    def _issue(i):
        r0, nrows = _CHUNKS[i]
        if nrows == 16:
            xb, xtb, sem = bufs[i % 2]
        else:
            xb, xtb, sem = x4buf, xt4buf, bufs[i % 2][2]
        h1 = pltpu.async_copy(sf_hbm.at[wid, pl.ds(r0, nrows), :], xb, sem)
        h2 = pltpu.async_copy(sft_hbm.at[wid, pl.ds(r0, nrows), :], xtb, sem)
        return h1, h2, xb, xtb

    pend = _issue(0)
    for i, (r0, nrows) in enumerate(_CHUNKS):
        h1, h2, xb, xtb = pend
        nxt = _issue(i + 1) if i + 1 < len(_CHUNKS) else None
        h1.wait()
        h2.wait()

        def _row_step(rr, carry):
            z = jnp.zeros((16,), jnp.float32)
            s_p, s_t, dot = z, z, z
            for j in range(_C // 16):
                v = xb[rr, pl.ds(16 * j, 16)]
                vt = xtb[rr, pl.ds(16 * j, 16)]
                w = jnp.exp(vt)
                s_p = s_p + jnp.exp(v)
                s_t = s_t + w
                dot = dot + w * (vt - v)
            spbuf[r0 + rr, :] = s_p
            stbuf[r0 + rr, :] = s_t
            dtbuf[r0 + rr, :] = dot
            return carry

        lax.fori_loop(0, nrows, _row_step, jnp.int32(0))
        pend = nxt

    pltpu.sync_copy(spbuf, sp_hbm.at[wid])
    pltpu.sync_copy(stbuf, st_hbm.at[wid])
    pltpu.sync_copy(dtbuf, dt_hbm.at[wid])


def _sc_rows(sf, sft):
    mesh = plsc.VectorSubcoreMesh(core_axis_name="c", subcore_axis_name="s")
    out = jax.ShapeDtypeStruct((_B_SC, _N, 16), jnp.float32)
    f = pl.kernel(
        _sc_body,
        mesh=mesh,
        out_type=[out, out, out],
        scratch_types=[pltpu.VMEM((16, _C), jnp.float32),
                       pltpu.VMEM((16, _C), jnp.float32),
                       pltpu.VMEM((16, _C), jnp.float32),
                       pltpu.VMEM((16, _C), jnp.float32),
                       pltpu.VMEM((4, _C), jnp.float32),
                       pltpu.VMEM((4, _C), jnp.float32),
                       pltpu.VMEM((_N, 16), jnp.float32),
                       pltpu.VMEM((_N, 16), jnp.float32),
                       pltpu.VMEM((_N, 16), jnp.float32),
                       pltpu.SemaphoreType.DMA,
                       pltpu.SemaphoreType.DMA],
    )
    return f(sf, sft)


def _row_kl_sum_terms(x, xt):
    mx = jnp.max(x, axis=-1, keepdims=True)
    mxt = jnp.max(xt, axis=-1, keepdims=True)
    s_p = jnp.sum(jnp.exp(x - mx), axis=-1)
    w = jnp.exp(xt - mxt)
    s_t = jnp.sum(w, axis=-1)
    dot = jnp.sum(w * (xt - x), axis=-1)
    return dot / s_t - (mxt - mx)[..., 0] + jnp.log(s_p / s_t)


def _sortable_key(x):
    b = lax.bitcast_convert_type(x, jnp.int32)
    sign = lax.shift_right_arithmetic(b, 31)
    return lax.bitwise_xor(b, lax.bitwise_and(sign, jnp.int32(0x7FFFFFFF)))


def _attn_sq_sum(s, m):
    key = _sortable_key(s)
    kth = jnp.full(s.shape[:-1] + (1,), 0, jnp.int32)

    def bit_step(i, prefix_u):
        bit = lax.shift_left(jnp.int32(1), 31 - i)
        cand_u = lax.bitwise_or(prefix_u, bit)
        cand_s = lax.bitwise_xor(cand_u, jnp.int32(_MININT))
        cnt = jnp.sum((key >= cand_s).astype(jnp.int32), axis=-1, keepdims=True)
        return jnp.where(cnt >= _K, cand_u, prefix_u)

    thr_u = lax.fori_loop(0, 32, bit_step, kth)
    thr_s = lax.bitwise_xor(thr_u, jnp.int32(_MININT))
    target = (key < thr_s).astype(jnp.float32)
    d = target - m
    return jnp.sum(d * d)


def _tc_body(pred_ref, predt_ref, s_ref, m_ref, sf_ref, sft_ref, ld_ref,
             out_ref, acc_ref):
    g = pl.program_id(0)

    @pl.when(g == 0)
    def _init():
        row = _row_kl_sum_terms(pred_ref[...], predt_ref[...])
        acc_ref[0] = jnp.sum(row)
        acc_ref[1] = _attn_sq_sum(s_ref[...], m_ref[...])
        acc_ref[2] = 0.0
        acc_ref[3] = 0.0
        acc_ref[4] = 0.0

    row_kl = _row_kl_sum_terms(sf_ref[...], sft_ref[...])
    ld = ld_ref[...]
    keep = ld > 0.5
    acc_ref[2] += jnp.sum(jnp.where(keep, row_kl, 0.0))
    acc_ref[3] += jnp.sum(keep.astype(jnp.float32))
    acc_ref[4] += jnp.sum(ld)

    @pl.when(g == _G - 1)
    def _fin():
        for i in range(5):
            out_ref[i] = acc_ref[i]


def _tc_main(pred, pred_t, s_stack, m_stack, sf, sft, ld):
    off = _B_SC // _BB
    return pl.pallas_call(
        _tc_body,
        grid=(_G,),
        in_specs=[
            pl.BlockSpec((_B, _NCLS), lambda g: (0, 0)),
            pl.BlockSpec((_B, _NCLS), lambda g: (0, 0)),
            pl.BlockSpec((3, _B, _N), lambda g: (0, 0, 0)),
            pl.BlockSpec((3, _B, _N), lambda g: (0, 0, 0)),
            pl.BlockSpec((_BB, _N, _C), lambda g: (off + g, 0, 0)),
            pl.BlockSpec((_BB, _N, _C), lambda g: (off + g, 0, 0)),
            pl.BlockSpec((_BB, _N), lambda g: (off + g, 0)),
        ],
        out_specs=pl.BlockSpec(memory_space=pltpu.SMEM),
        out_shape=jax.ShapeDtypeStruct((8,), jnp.float32),
        scratch_shapes=[pltpu.SMEM((8,), jnp.float32)],
    )(pred, pred_t, s_stack, m_stack, sf, sft, ld)


def _comb_body(acc_ref, sp_ref, st_ref, dt_ref, ld_ref, out_ref):
    sp16 = sp_ref[...]
    st16 = st_ref[...]
    dt16 = dt_ref[...]
    sp = jnp.sum(sp16, axis=-1)
    st = jnp.sum(st16, axis=-1)
    dt = jnp.sum(dt16, axis=-1)
    row_kl = dt / st + jnp.log(sp / st)
    ld = ld_ref[0:_B_SC, :]
    keep = ld > 0.5
    kl_sum = acc_ref[2] + jnp.sum(jnp.where(keep, row_kl, 0.0))
    cnt = acc_ref[3] + jnp.sum(keep.astype(jnp.float32))
    ld_sum = acc_ref[4] + jnp.sum(ld)
    attn = (2.0 / 3.0) * acc_ref[1] / (_B * _N)
    cls_kl = acc_ref[0] / _B
    token = jnp.where(ld_sum < 0.1, 0.0, kl_sum / cnt)
    total = attn + 0.5 * cls_kl + 0.5 * token
    out_ref[...] = jnp.broadcast_to(total, (1, 1))


def _combine(accs, sc_sp, sc_st, sc_dt, ld):
    return pl.pallas_call(
        _comb_body,
        in_specs=[
            pl.BlockSpec(memory_space=pltpu.SMEM),
            pl.BlockSpec((_B_SC, _N, 16), lambda: (0, 0, 0)),
            pl.BlockSpec((_B_SC, _N, 16), lambda: (0, 0, 0)),
            pl.BlockSpec((_B_SC, _N, 16), lambda: (0, 0, 0)),
            pl.BlockSpec((_B, _N), lambda: (0, 0)),
        ],
        out_specs=pl.BlockSpec((1, 1), lambda: (0, 0)),
        out_shape=jax.ShapeDtypeStruct((1, 1), jnp.float32),
    )(accs, sc_sp, sc_st, sc_dt, ld)


def kernel(pred, pred_t, spatial_features, last_decision, spatial_features_t,
           hard_keep_decision_0, hard_keep_decision_1, hard_keep_decision_2,
           token_attn_sim_0, token_attn_sim_1, token_attn_sim_2):
    s_stack = jnp.stack([token_attn_sim_0[:, :, 1],
                         token_attn_sim_1[:, :, 1],
                         token_attn_sim_2[:, :, 1]])
    m_stack = jnp.stack([hard_keep_decision_0, hard_keep_decision_1,
                         hard_keep_decision_2])

    sc_sp, sc_st, sc_dt = _sc_rows(spatial_features, spatial_features_t)
    accs = _tc_main(pred, pred_t, s_stack, m_stack,
                    spatial_features, spatial_features_t, last_decision)
    out = _combine(accs, sc_sp, sc_st, sc_dt, last_decision)
    return out.reshape(())


# speedup vs baseline: 1.0168x; 1.0168x over previous
import jax
import jax.numpy as jnp
from jax import lax
from jax.experimental import pallas as pl
from jax.experimental.pallas import tpu as pltpu
from jax.experimental.pallas import tpu_sc as plsc

_B, _N, _C, _NCLS = 64, 196, 384, 1000
_K = int((1.0 - 0.7) * _N)
_MININT = -2147483648

_B_SC = 32
_R_SC = _B_SC * _N
_B_TC = _B - _B_SC
_BB = 8
_G = _B_TC // _BB
_CHUNKS = [(r0, 16) for r0 in range(0, 192, 16)] + [(192, 4)]


def _sc_body(sf_hbm, sft_hbm, sp_hbm, st_hbm, dt_hbm,
             xbuf0, xtbuf0, xbuf1, xtbuf1, x4buf, xt4buf,
             spbuf, stbuf, dtbuf, sem0, sem1):
    nc = 2
    wid = lax.axis_index("s") * nc + lax.axis_index("c")
    bufs = [(xbuf0, xtbuf0, sem0), (xbuf1, xtbuf1, sem1)]

    @pl.when(wid < _B_SC)
    def _active():
        _sc_tile(sf_hbm, sft_hbm, sp_hbm, st_hbm, dt_hbm, wid, bufs,
                 x4buf, xt4buf, spbuf, stbuf, dtbuf)


def _sc_tile(sf_hbm, sft_hbm, sp_hbm, st_hbm, dt_hbm, wid, bufs,
             x4buf, xt4buf, spbuf, stbuf, dtbuf):

    def _issue(i):
        r0, nrows = _CHUNKS[i]
        if nrows == 16:
            xb, xtb, sem = bufs[i % 2]
        else:
            xb, xtb, sem = x4buf, xt4buf, bufs[i % 2][2]
        h1 = pltpu.async_copy(sf_hbm.at[wid, pl.ds(r0, nrows), :], xb, sem)
        h2 = pltpu.async_copy(sft_hbm.at[wid, pl.ds(r0, nrows), :], xtb, sem)
        return h1, h2, xb, xtb

    pend = _issue(0)
    for i, (r0, nrows) in enumerate(_CHUNKS):
        h1, h2, xb, xtb = pend
        nxt = _issue(i + 1) if i + 1 < len(_CHUNKS) else None
        h1.wait()
        h2.wait()

        def _row_step(rr, carry):
            z = jnp.zeros((16,), jnp.float32)
            s_p, s_t, dot = z, z, z
            for j in range(_C // 16):
                v = xb[rr, pl.ds(16 * j, 16)]
                vt = xtb[rr, pl.ds(16 * j, 16)]
                w = jnp.exp(vt)
                s_p = s_p + jnp.exp(v)
                s_t = s_t + w
                dot = dot + w * (vt - v)
            spbuf[r0 + rr, :] = s_p
            stbuf[r0 + rr, :] = s_t
            dtbuf[r0 + rr, :] = dot
            return carry

        lax.fori_loop(0, nrows, _row_step, jnp.int32(0))
        pend = nxt

    pltpu.sync_copy(spbuf, sp_hbm.at[wid])
    pltpu.sync_copy(stbuf, st_hbm.at[wid])
    pltpu.sync_copy(dtbuf, dt_hbm.at[wid])


def _sc_rows(sf, sft):
    mesh = plsc.VectorSubcoreMesh(core_axis_name="c", subcore_axis_name="s")
    out = jax.ShapeDtypeStruct((_B_SC, _N, 16), jnp.float32)
    f = pl.kernel(
        _sc_body,
        mesh=mesh,
        out_type=[out, out, out],
        scratch_types=[pltpu.VMEM((16, _C), jnp.float32),
                       pltpu.VMEM((16, _C), jnp.float32),
                       pltpu.VMEM((16, _C), jnp.float32),
                       pltpu.VMEM((16, _C), jnp.float32),
                       pltpu.VMEM((4, _C), jnp.float32),
                       pltpu.VMEM((4, _C), jnp.float32),
                       pltpu.VMEM((_N, 16), jnp.float32),
                       pltpu.VMEM((_N, 16), jnp.float32),
                       pltpu.VMEM((_N, 16), jnp.float32),
                       pltpu.SemaphoreType.DMA,
                       pltpu.SemaphoreType.DMA],
    )
    return f(sf, sft)


def _row_kl_sum_terms(x, xt):
    mx = jnp.max(x, axis=-1, keepdims=True)
    mxt = jnp.max(xt, axis=-1, keepdims=True)
    s_p = jnp.sum(jnp.exp(x - mx), axis=-1)
    w = jnp.exp(xt - mxt)
    s_t = jnp.sum(w, axis=-1)
    dot = jnp.sum(w * (xt - x), axis=-1)
    return dot / s_t - (mxt - mx)[..., 0] + jnp.log(s_p / s_t)


def _sortable_key(x):
    b = lax.bitcast_convert_type(x, jnp.int32)
    sign = lax.shift_right_arithmetic(b, 31)
    return lax.bitwise_xor(b, lax.bitwise_and(sign, jnp.int32(0x7FFFFFFF)))


def _attn_sq_sum(s, m):
    key = _sortable_key(s)
    kth = jnp.full(s.shape[:-1] + (1,), 0, jnp.int32)

    def bit_step(i, prefix_u):
        bit = lax.shift_left(jnp.int32(1), 31 - i)
        cand_u = lax.bitwise_or(prefix_u, bit)
        cand_s = lax.bitwise_xor(cand_u, jnp.int32(_MININT))
        cnt = jnp.sum((key >= cand_s).astype(jnp.int32), axis=-1, keepdims=True)
        return jnp.where(cnt >= _K, cand_u, prefix_u)

    thr_u = lax.fori_loop(0, 32, bit_step, kth)
    thr_s = lax.bitwise_xor(thr_u, jnp.int32(_MININT))
    target = (key < thr_s).astype(jnp.float32)
    d = target - m
    return jnp.sum(d * d)


def _tc_body(pred_ref, predt_ref, s_ref, m_ref, sf_ref, sft_ref, ld_ref,
             out_ref, acc_ref):
    g = pl.program_id(0)

    @pl.when(g == 0)
    def _init():
        row = _row_kl_sum_terms(pred_ref[...], predt_ref[...])
        acc_ref[0] = jnp.sum(row)
        acc_ref[1] = _attn_sq_sum(s_ref[...], m_ref[...])
        acc_ref[2] = 0.0
        acc_ref[3] = 0.0
        acc_ref[4] = 0.0

    row_kl = _row_kl_sum_terms(sf_ref[...], sft_ref[...])
    ld = ld_ref[...]
    keep = ld > 0.5
    acc_ref[2] += jnp.sum(jnp.where(keep, row_kl, 0.0))
    acc_ref[3] += jnp.sum(keep.astype(jnp.float32))
    acc_ref[4] += jnp.sum(ld)

    @pl.when(g == _G - 1)
    def _fin():
        for i in range(5):
            out_ref[i] = acc_ref[i]


def _tc_main(pred, pred_t, s_stack, m_stack, sf, sft, ld):
    off = _B_SC // _BB
    return pl.pallas_call(
        _tc_body,
        grid=(_G,),
        in_specs=[
            pl.BlockSpec((_B, _NCLS), lambda g: (0, 0)),
            pl.BlockSpec((_B, _NCLS), lambda g: (0, 0)),
            pl.BlockSpec((3, _B, _N), lambda g: (0, 0, 0)),
            pl.BlockSpec((3, _B, _N), lambda g: (0, 0, 0)),
            pl.BlockSpec((_BB, _N, _C), lambda g: (off + g, 0, 0)),
            pl.BlockSpec((_BB, _N, _C), lambda g: (off + g, 0, 0)),
            pl.BlockSpec((_BB, _N), lambda g: (off + g, 0)),
        ],
        out_specs=pl.BlockSpec(memory_space=pltpu.SMEM),
        out_shape=jax.ShapeDtypeStruct((8,), jnp.float32),
        scratch_shapes=[pltpu.SMEM((8,), jnp.float32)],
    )(pred, pred_t, s_stack, m_stack, sf, sft, ld)


def _comb_body(acc_ref, sp_ref, st_ref, dt_ref, ld_ref, out_ref):
    sp16 = sp_ref[...]
    st16 = st_ref[...]
    dt16 = dt_ref[...]
    sp = jnp.sum(sp16, axis=-1)
    st = jnp.sum(st16, axis=-1)
    dt = jnp.sum(dt16, axis=-1)
    row_kl = dt / st + jnp.log(sp / st)
    ld = ld_ref[0:_B_SC, :]
    keep = ld > 0.5
    kl_sum = acc_ref[2] + jnp.sum(jnp.where(keep, row_kl, 0.0))
    cnt = acc_ref[3] + jnp.sum(keep.astype(jnp.float32))
    ld_sum = acc_ref[4] + jnp.sum(ld)
    attn = (2.0 / 3.0) * acc_ref[1] / (_B * _N)
    cls_kl = acc_ref[0] / _B
    token = jnp.where(ld_sum < 0.1, 0.0, kl_sum / cnt)
    total = attn + 0.5 * cls_kl + 0.5 * token
    out_ref[...] = jnp.broadcast_to(total, (1, 1))


def _combine(accs, sc_sp, sc_st, sc_dt, ld):
    return pl.pallas_call(
        _comb_body,
        in_specs=[
            pl.BlockSpec(memory_space=pltpu.SMEM),
            pl.BlockSpec((_B_SC, _N, 16), lambda: (0, 0, 0)),
            pl.BlockSpec((_B_SC, _N, 16), lambda: (0, 0, 0)),
            pl.BlockSpec((_B_SC, _N, 16), lambda: (0, 0, 0)),
            pl.BlockSpec((_B, _N), lambda: (0, 0)),
        ],
        out_specs=pl.BlockSpec((1, 1), lambda: (0, 0)),
        out_shape=jax.ShapeDtypeStruct((1, 1), jnp.float32),
    )(accs, sc_sp, sc_st, sc_dt, ld)


def kernel(pred, pred_t, spatial_features, last_decision, spatial_features_t,
           hard_keep_decision_0, hard_keep_decision_1, hard_keep_decision_2,
           token_attn_sim_0, token_attn_sim_1, token_attn_sim_2):
    s_stack = jnp.stack([token_attn_sim_0[:, :, 1],
                         token_attn_sim_1[:, :, 1],
                         token_attn_sim_2[:, :, 1]])
    m_stack = jnp.stack([hard_keep_decision_0, hard_keep_decision_1,
                         hard_keep_decision_2])

    sc_sp, sc_st, sc_dt = _sc_rows(spatial_features, spatial_features_t)
    accs = _tc_main(pred, pred_t, s_stack, m_stack,
                    spatial_features, spatial_features_t, last_decision)
    out = _combine(accs, sc_sp, sc_st, sc_dt, last_decision)
    return out.reshape(())
